# baseline (speedup 1.0000x reference)
import functools

import jax
import jax.numpy as jnp
from jax.experimental import pallas as pl

F32 = jnp.float32


def _mm_kern(relu, x_ref, w_ref, b_ref, o_ref):
    acc = jnp.dot(x_ref[...], w_ref[...], preferred_element_type=F32)
    acc = acc + b_ref[...]
    if relu:
        acc = jnp.maximum(acc, 0.0)
    o_ref[...] = acc


def _matmul(x, w, b, relu, tm):
    m, k = x.shape
    n = w.shape[1]
    mp = ((m + tm - 1) // tm) * tm
    if mp != m:
        x = jnp.pad(x, ((0, mp - m), (0, 0)))
    out = pl.pallas_call(
        functools.partial(_mm_kern, relu),
        grid=(mp // tm,),
        in_specs=[
            pl.BlockSpec((tm, k), lambda i: (i, 0)),
            pl.BlockSpec((k, n), lambda i: (0, 0)),
            pl.BlockSpec((1, n), lambda i: (0, 0)),
        ],
        out_specs=pl.BlockSpec((tm, n), lambda i: (i, 0)),
        out_shape=jax.ShapeDtypeStruct((mp, n), F32),
    )(x, w, b.reshape(1, n))
    return out[:m] if mp != m else out


def _conv1x1(x, w, b, relu, tm=1024):
    bsz, h, wd, c = x.shape
    co = w.shape[-1]
    y = _matmul(x.reshape(-1, c), w.reshape(c, co), b, relu, tm)
    return y.reshape(bsz, h, wd, co)


def _c3_kern(th, ow, dil, relu, x_ref, w_ref, b_ref, o_ref):
    c = x_ref.shape[-1]
    tco = w_ref.shape[-1]
    acc = jnp.zeros((th * ow, tco), F32)
    for dh in range(3):
        for dw in range(3):
            sl = x_ref[0, 0, dh * dil:dh * dil + th, dw * dil:dw * dil + ow, :]
            acc = acc + jnp.dot(sl.reshape(th * ow, c), w_ref[dh, dw],
                                preferred_element_type=F32)
    acc = acc + b_ref[...]
    if relu:
        acc = jnp.maximum(acc, 0.0)
    o_ref[0, 0] = acc.reshape(th, ow, tco)


def _conv3(x, w, b, pad, dil=1, relu=True, nt=1, tco=None):
    bsz, h, wd, c = x.shape
    co = w.shape[-1]
    ek = 2 * dil + 1
    xp = jnp.pad(x, ((0, 0), (pad, pad), (pad, pad), (0, 0)))
    ph, pw = h + 2 * pad, wd + 2 * pad
    oh, ow = ph - ek + 1, pw - ek + 1
    th = oh // nt
    wh = th + ek - 1
    if nt == 1:
        xw = xp[:, None]
    else:
        xw = jnp.stack(
            [jax.lax.slice_in_dim(xp, t * th, t * th + wh, axis=1)
             for t in range(nt)], axis=1)
    tco = tco or co
    nco = co // tco
    out = pl.pallas_call(
        functools.partial(_c3_kern, th, ow, dil, relu),
        grid=(bsz, nt, nco),
        in_specs=[
            pl.BlockSpec((1, 1, wh, pw, c), lambda bb, t, cc: (bb, t, 0, 0, 0)),
            pl.BlockSpec((3, 3, c, tco), lambda bb, t, cc: (0, 0, 0, cc)),
            pl.BlockSpec((1, tco), lambda bb, t, cc: (0, cc)),
        ],
        out_specs=pl.BlockSpec((1, 1, th, ow, tco),
                               lambda bb, t, cc: (bb, t, 0, 0, cc)),
        out_shape=jax.ShapeDtypeStruct((bsz, nt, th, ow, co), F32),
    )(xw, w, b.reshape(1, co))
    return out.reshape(bsz, oh, ow, co)


def _c3s2_kern(oh, ow, relu, x_ref, w_ref, b_ref, o_ref):
    c = x_ref.shape[-1]
    tco = w_ref.shape[-1]
    acc = jnp.zeros((oh * ow, tco), F32)
    for dh in range(3):
        for dw in range(3):
            r0, c0 = dh // 2, dw // 2
            sl = x_ref[0, dh % 2, dw % 2, r0:r0 + oh, c0:c0 + ow, :]
            acc = acc + jnp.dot(sl.reshape(oh * ow, c), w_ref[dh, dw],
                                preferred_element_type=F32)
    acc = acc + b_ref[...]
    if relu:
        acc = jnp.maximum(acc, 0.0)
    o_ref[0] = acc.reshape(oh, ow, tco)


def _conv3_s2(x, w, b, pad, relu=True):
    bsz, h, wd, c = x.shape
    co = w.shape[-1]
    oh = (h + 2 * pad - 3) // 2 + 1
    ph2 = 2 * (oh + 1)
    xp = jnp.pad(x, ((0, 0), (pad, ph2 - h - pad), (pad, ph2 - wd - pad),
                     (0, 0)))
    phases = jnp.stack(
        [jnp.stack([xp[:, a::2, b2::2] for b2 in (0, 1)], axis=1)
         for a in (0, 1)], axis=1)
    hp = ph2 // 2
    out = pl.pallas_call(
        functools.partial(_c3s2_kern, oh, oh, relu),
        grid=(bsz,),
        in_specs=[
            pl.BlockSpec((1, 2, 2, hp, hp, c), lambda bb: (bb, 0, 0, 0, 0, 0)),
            pl.BlockSpec((3, 3, c, co), lambda bb: (0, 0, 0, 0)),
            pl.BlockSpec((1, co), lambda bb: (0, 0)),
        ],
        out_specs=pl.BlockSpec((1, oh, oh, co), lambda bb: (bb, 0, 0, 0)),
        out_shape=jax.ShapeDtypeStruct((bsz, oh, oh, co), F32),
    )(phases, w, b.reshape(1, co))
    return out


def _max_kern(p, x_ref, o_ref):
    m = x_ref[0, 0]
    for i in range(1, p):
        m = jnp.maximum(m, x_ref[0, i])
    o_ref[0] = m


def _max_reduce(stk, nt=1):
    bsz, p, h, wd, c = stk.shape
    th = h // nt
    return pl.pallas_call(
        functools.partial(_max_kern, p),
        grid=(bsz, nt),
        in_specs=[pl.BlockSpec((1, p, th, wd, c),
                               lambda bb, t: (bb, 0, t, 0, 0))],
        out_specs=pl.BlockSpec((1, th, wd, c), lambda bb, t: (bb, t, 0, 0)),
        out_shape=jax.ShapeDtypeStruct((bsz, h, wd, c), F32),
    )(stk)


def _maxpool2(x, ceil=False, nt=1):
    bsz, h, wd, c = x.shape
    if ceil and h % 2:
        x = jnp.pad(x, ((0, 0), (0, 1), (0, 1), (0, 0)),
                    constant_values=-jnp.inf)
        h, wd = h + 1, wd + 1
    stk = jnp.stack([x[:, a::2, b2::2] for a in (0, 1) for b2 in (0, 1)],
                    axis=1)
    return _max_reduce(stk, nt)


def _maxpool3_s1(x):
    bsz, h, wd, c = x.shape
    xp = jnp.pad(x, ((0, 0), (1, 1), (1, 1), (0, 0)),
                 constant_values=-jnp.inf)
    stk = jnp.stack([xp[:, a:a + h, b2:b2 + wd] for a in range(3)
                     for b2 in range(3)], axis=1)
    return _max_reduce(stk)


def _l2_kern(x_ref, r_ref, o_ref):
    x = x_ref[0]
    ss = jnp.sum(x * x, axis=-1, keepdims=True)
    o_ref[0] = x / jnp.sqrt(ss) * r_ref[...]


def _l2_rescale(x, r):
    bsz, h, wd, c = x.shape
    return pl.pallas_call(
        _l2_kern,
        grid=(bsz,),
        in_specs=[
            pl.BlockSpec((1, h, wd, c), lambda bb: (bb, 0, 0, 0)),
            pl.BlockSpec((1, c), lambda bb: (0, 0)),
        ],
        out_specs=pl.BlockSpec((1, h, wd, c), lambda bb: (bb, 0, 0, 0)),
        out_shape=jax.ShapeDtypeStruct((bsz, h, wd, c), F32),
    )(x, r.reshape(1, c))


def kernel(image, params):
    p = params
    w = {k[:-2]: jnp.transpose(v, (2, 3, 1, 0))
         for k, v in p.items() if k.endswith('_w')}

    x = jnp.transpose(image, (0, 2, 3, 1))
    bsz = x.shape[0]

    xp = jnp.pad(x, ((0, 0), (1, 1), (1, 1), (0, 0)))
    cols = jnp.concatenate(
        [xp[:, a:a + 300, b2:b2 + 300, :] for a in range(3) for b2 in range(3)],
        axis=-1)
    x = _matmul(cols.reshape(-1, 27), w['c1_1'].reshape(27, 64),
                p['c1_1_b'], True, tm=4096).reshape(bsz, 300, 300, 64)

    x = _conv3(x, w['c1_2'], p['c1_2_b'], pad=1, nt=6)
    x = _maxpool2(x, nt=3)
    x = _conv3(x, w['c2_1'], p['c2_1_b'], pad=1, nt=3)
    x = _conv3(x, w['c2_2'], p['c2_2_b'], pad=1, nt=3)
    x = _maxpool2(x, nt=3)
    x = _conv3(x, w['c3_1'], p['c3_1_b'], pad=1, nt=3)
    x = _conv3(x, w['c3_2'], p['c3_2_b'], pad=1, nt=3)
    x = _conv3(x, w['c3_3'], p['c3_3_b'], pad=1, nt=3)
    x = _maxpool2(x, ceil=True, nt=2)
    x = _conv3(x, w['c4_1'], p['c4_1_b'], pad=1)
    x = _conv3(x, w['c4_2'], p['c4_2_b'], pad=1, tco=256)
    x = _conv3(x, w['c4_3'], p['c4_3_b'], pad=1, tco=256)
    c4 = x
    x = _maxpool2(x)
    x = _conv3(x, w['c5_1'], p['c5_1_b'], pad=1, tco=256)
    x = _conv3(x, w['c5_2'], p['c5_2_b'], pad=1, tco=256)
    x = _conv3(x, w['c5_3'], p['c5_3_b'], pad=1, tco=256)
    x = _maxpool3_s1(x)
    x = _conv3(x, w['c6'], p['c6_b'], pad=6, dil=6, tco=256)
    c7 = _conv1x1(x, w['c7'], p['c7_b'], True)

    c4 = _l2_rescale(c4, p['rescale'].reshape(-1))

    x = _conv1x1(c7, w['c8_1'], p['c8_1_b'], True)
    c8 = _conv3_s2(x, w['c8_2'], p['c8_2_b'], pad=1)
    x = _conv1x1(c8, w['c9_1'], p['c9_1_b'], True)
    c9 = _conv3_s2(x, w['c9_2'], p['c9_2_b'], pad=1)
    x = _conv1x1(c9, w['c10_1'], p['c10_1_b'], True)
    c10 = _conv3(x, w['c10_2'], p['c10_2_b'], pad=0)
    x = _conv1x1(c10, w['c11_1'], p['c11_1_b'], True)
    c11 = _conv3(x, w['c11_2'], p['c11_2_b'], pad=0)

    def head(feat, name, d):
        y = _conv3(feat, w[name], p[name + '_b'], pad=1, relu=False)
        return y.reshape(bsz, -1, d)

    feats = [(c4, '4'), (c7, '7'), (c8, '8'), (c9, '9'), (c10, '10'),
             (c11, '11')]
    locs = jnp.concatenate([head(f, 'loc' + s, 4) for f, s in feats], axis=1)
    nc = p['cl4_w'].shape[0] // 4
    cls = jnp.concatenate([head(f, 'cl' + s, nc) for f, s in feats], axis=1)
    return (locs, cls)
